# Initial kernel scaffold; baseline (speedup 1.0000x reference)
import jax
import jax.numpy as jnp
from jax.experimental import pallas as pl


def kernel(indices, v_table, mask_table):
    raise NotImplementedError("write your pallas kernel here")


# speedup vs baseline: 1.2331x; 1.2331x over previous
import functools

import jax
import jax.numpy as jnp
from jax import lax
from jax.experimental import pallas as pl
from jax.experimental.pallas import tpu as pltpu
from jax.experimental.pallas import tpu_sc as plsc

N_OBJ = 11
PAD_LEN = 4000
BATCH = 1024
NROW = PAD_LEN * 3
TW = 16

_info = plsc.get_sparse_core_info()
_NC = _info.num_cores
_NS = _info.num_subcores
_NW = _NC * _NS

_RC = 32
_N_CHUNKS = NROW // _RC
_CPW = -(-_N_CHUNKS // _NW)
_NIV = BATCH // 16


def _premul_body(v_ref, m_ref, o_ref):
    o_ref[...] = v_ref[...] * m_ref[...]


def _premul(v16, m16):
    return pl.pallas_call(
        _premul_body,
        grid=(3,),
        in_specs=[
            pl.BlockSpec((PAD_LEN, TW), lambda i: (i, 0)),
            pl.BlockSpec((PAD_LEN, TW), lambda i: (0, 0)),
        ],
        out_specs=pl.BlockSpec((PAD_LEN, TW), lambda i: (i, 0)),
        out_shape=jax.ShapeDtypeStruct((NROW, TW), jnp.float32),
    )(v16, m16)


@functools.partial(
    pl.kernel,
    mesh=plsc.VectorSubcoreMesh(core_axis_name="c", subcore_axis_name="s"),
    out_type=jax.ShapeDtypeStruct((NROW, BATCH), jnp.float32),
    scratch_types=[
        pltpu.VMEM((BATCH,), jnp.int32),
        pltpu.VMEM((2, _RC, TW), jnp.float32),
        pltpu.VMEM((2, _RC, BATCH), jnp.float32),
        pltpu.SemaphoreType.DMA,
        pltpu.SemaphoreType.DMA,
    ],
)
def _sc_gather(table_hbm, idx_hbm, out_hbm, idx_v, tbl_v, obuf, gsem, ssem):
    wid = lax.axis_index("s") * _NC + lax.axis_index("c")
    pltpu.sync_copy(idx_hbm, idx_v)

    def do_chunk(c, b):
        gchunk = c * _NW + wid
        valid = gchunk < _N_CHUNKS
        row0 = pl.multiple_of(gchunk * _RC, _RC)

        @pl.when(valid & (c >= 2))
        def _wait_prev_scatter():
            pltpu.make_async_copy(
                obuf.at[b], out_hbm.at[pl.ds(0, _RC)], ssem
            ).wait()

        @pl.when(valid)
        def _work():
            pltpu.async_copy(
                table_hbm.at[pl.ds(row0, _RC)], tbl_v.at[b], gsem
            ).wait()

            def row_body(r, _):
                row = tbl_v.at[b][r]
                for iv in range(_NIV):
                    ivec = idx_v[pl.ds(iv * 16, 16)]
                    g = row.at[ivec].get(mode="promise_in_bounds")
                    obuf.at[b][r, pl.ds(iv * 16, 16)] = g
                return _

            lax.fori_loop(0, _RC, row_body, 0, unroll=False)
            pltpu.async_copy(
                obuf.at[b], out_hbm.at[pl.ds(row0, _RC)], ssem
            )

    def step(k, _):
        do_chunk(k * 2, 0)
        do_chunk(k * 2 + 1, 1)
        return _

    lax.fori_loop(0, _CPW // 2, step, 0, unroll=False)

    for b in range(2):
        pltpu.make_async_copy(
            obuf.at[b], out_hbm.at[pl.ds(0, _RC)], ssem
        ).wait()


def kernel(indices, v_table, mask_table):
    idx = indices.astype(jnp.int32)
    vT = v_table.transpose(2, 1, 0).reshape(NROW, N_OBJ)
    v16 = jnp.pad(vT, ((0, 0), (0, TW - N_OBJ)))
    m16 = jnp.pad(mask_table.T, ((0, 0), (0, TW - N_OBJ)))
    mvT = _premul(v16, m16)
    out2d = _sc_gather(mvT, idx)
    return out2d.reshape(3, PAD_LEN, BATCH).transpose(2, 1, 0)


# speedup vs baseline: 2.5547x; 2.0718x over previous
import functools

import jax
import jax.numpy as jnp
from jax import lax
from jax.experimental import pallas as pl
from jax.experimental.pallas import tpu as pltpu
from jax.experimental.pallas import tpu_sc as plsc

N_OBJ = 11
PAD_LEN = 4000
BATCH = 1024
NROW = PAD_LEN * 3
TW = 16

_info = plsc.get_sparse_core_info()
_NC = _info.num_cores
_NS = _info.num_subcores
_NW = _NC * _NS

_RC = 32
_N_CHUNKS = NROW // _RC
_CPW = -(-_N_CHUNKS // _NW)
_NIV = BATCH // 16
_IVB = 8


def _premul_body(v_ref, m_ref, o_ref):
    o_ref[...] = v_ref[...] * m_ref[...]


def _premul(v16, m16):
    return pl.pallas_call(
        _premul_body,
        grid=(3,),
        in_specs=[
            pl.BlockSpec((PAD_LEN, TW), lambda i: (i, 0)),
            pl.BlockSpec((PAD_LEN, TW), lambda i: (0, 0)),
        ],
        out_specs=pl.BlockSpec((PAD_LEN, TW), lambda i: (i, 0)),
        out_shape=jax.ShapeDtypeStruct((NROW, TW), jnp.float32),
    )(v16, m16)


@functools.partial(
    pl.kernel,
    mesh=plsc.VectorSubcoreMesh(core_axis_name="c", subcore_axis_name="s"),
    out_type=jax.ShapeDtypeStruct((NROW, BATCH), jnp.float32),
    scratch_types=[
        pltpu.VMEM((BATCH,), jnp.int32),
        pltpu.VMEM((2, _RC, TW), jnp.float32),
        pltpu.VMEM((2, _RC, BATCH), jnp.float32),
        pltpu.SemaphoreType.DMA,
        pltpu.SemaphoreType.DMA,
    ],
)
def _sc_gather(table_hbm, idx_hbm, out_hbm, idx_v, tbl_v, obuf, gsem, ssem):
    wid = lax.axis_index("s") * _NC + lax.axis_index("c")
    pltpu.sync_copy(idx_hbm, idx_v)

    def do_chunk(c, b):
        gchunk = c * _NW + wid
        valid = gchunk < _N_CHUNKS
        row0 = pl.multiple_of(gchunk * _RC, _RC)

        @pl.when(valid & (c >= 2))
        def _wait_prev_scatter():
            pltpu.make_async_copy(
                obuf.at[b], out_hbm.at[pl.ds(0, _RC)], ssem
            ).wait()

        @pl.when(valid)
        def _work():
            pltpu.async_copy(
                table_hbm.at[pl.ds(row0, _RC)], tbl_v.at[b], gsem
            ).wait()

            for t in range(_NIV // _IVB):
                ivecs = [
                    idx_v[pl.ds((t * _IVB + j) * 16, 16)]
                    for j in range(_IVB)
                ]

                def row_body(r, _):
                    row = tbl_v.at[b][r]
                    for j in range(_IVB):
                        g = row.at[ivecs[j]].get(mode="promise_in_bounds")
                        obuf.at[b][r, pl.ds((t * _IVB + j) * 16, 16)] = g
                    return _

                lax.fori_loop(0, _RC, row_body, 0, unroll=False)
            pltpu.async_copy(
                obuf.at[b], out_hbm.at[pl.ds(row0, _RC)], ssem
            )

    def step(k, _):
        do_chunk(k * 2, 0)
        do_chunk(k * 2 + 1, 1)
        return _

    lax.fori_loop(0, _CPW // 2, step, 0, unroll=False)

    for b in range(2):
        pltpu.make_async_copy(
            obuf.at[b], out_hbm.at[pl.ds(0, _RC)], ssem
        ).wait()


def kernel(indices, v_table, mask_table):
    idx = indices.astype(jnp.int32)
    vT = v_table.transpose(2, 1, 0).reshape(NROW, N_OBJ)
    v16 = jnp.pad(vT, ((0, 0), (0, TW - N_OBJ)))
    m16 = jnp.pad(mask_table.T, ((0, 0), (0, TW - N_OBJ)))
    mvT = _premul(v16, m16)
    out2d = _sc_gather(mvT, idx)
    return out2d.reshape(3, PAD_LEN, BATCH).transpose(2, 1, 0)


# speedup vs baseline: 2.6839x; 1.0506x over previous
import functools

import jax
import jax.numpy as jnp
from jax import lax
from jax.experimental import pallas as pl
from jax.experimental.pallas import tpu as pltpu
from jax.experimental.pallas import tpu_sc as plsc

N_OBJ = 11
PAD_LEN = 4000
BATCH = 1024
NROW = PAD_LEN * 3
TW = 16

_info = plsc.get_sparse_core_info()
_NC = _info.num_cores
_NS = _info.num_subcores
_NW = _NC * _NS

_RC = 32
_N_CHUNKS = NROW // _RC
_CPW = -(-_N_CHUNKS // _NW)
_NIV = BATCH // 16
_IVB = 8


@functools.partial(
    pl.kernel,
    mesh=plsc.VectorSubcoreMesh(core_axis_name="c", subcore_axis_name="s"),
    out_type=jax.ShapeDtypeStruct((NROW, BATCH), jnp.float32),
    scratch_types=[
        pltpu.VMEM((BATCH,), jnp.int32),
        pltpu.VMEM((2, _RC, TW), jnp.float32),
        pltpu.VMEM((2, _RC, TW), jnp.float32),
        pltpu.VMEM((2, _RC, BATCH), jnp.float32),
        pltpu.SemaphoreType.DMA,
        pltpu.SemaphoreType.DMA,
        pltpu.SemaphoreType.DMA,
    ],
)
def _sc_gather(
    table_hbm, mask_hbm, idx_hbm, out_hbm,
    idx_v, tbl_v, msk_v, obuf, gsem, msem, ssem,
):
    wid = lax.axis_index("s") * _NC + lax.axis_index("c")
    pltpu.sync_copy(idx_hbm, idx_v)

    def do_chunk(c, b):
        gchunk = c * _NW + wid
        valid = gchunk < _N_CHUNKS
        row0 = pl.multiple_of(gchunk * _RC, _RC)

        @pl.when(valid & (c >= 2))
        def _wait_prev_scatter():
            pltpu.make_async_copy(
                obuf.at[b], out_hbm.at[pl.ds(0, _RC)], ssem
            ).wait()

        @pl.when(valid)
        def _work():
            pltpu.async_copy(
                table_hbm.at[pl.ds(row0, _RC)], tbl_v.at[b], gsem
            )
            mrow0 = pl.multiple_of(row0 - (row0 // PAD_LEN) * PAD_LEN, _RC)
            pltpu.async_copy(
                mask_hbm.at[pl.ds(mrow0, _RC)], msk_v.at[b], msem
            ).wait()
            pltpu.make_async_copy(
                table_hbm.at[pl.ds(row0, _RC)], tbl_v.at[b], gsem
            ).wait()

            def fold_body(r, _):
                tbl_v.at[b][r] = tbl_v.at[b][r] * msk_v.at[b][r]
                return _

            lax.fori_loop(0, _RC, fold_body, 0, unroll=False)

            for t in range(_NIV // _IVB):
                ivecs = [
                    idx_v[pl.ds((t * _IVB + j) * 16, 16)]
                    for j in range(_IVB)
                ]

                def row_body(r, _):
                    row = tbl_v.at[b][r]
                    for j in range(_IVB):
                        g = row.at[ivecs[j]].get(mode="promise_in_bounds")
                        obuf.at[b][r, pl.ds((t * _IVB + j) * 16, 16)] = g
                    return _

                lax.fori_loop(0, _RC, row_body, 0, unroll=False)
            pltpu.async_copy(
                obuf.at[b], out_hbm.at[pl.ds(row0, _RC)], ssem
            )

    def step(k, _):
        do_chunk(k * 2, 0)
        do_chunk(k * 2 + 1, 1)
        return _

    lax.fori_loop(0, _CPW // 2, step, 0, unroll=False)

    for b in range(2):
        pltpu.make_async_copy(
            obuf.at[b], out_hbm.at[pl.ds(0, _RC)], ssem
        ).wait()


def kernel(indices, v_table, mask_table):
    idx = indices.astype(jnp.int32)
    vT = v_table.transpose(2, 1, 0).reshape(NROW, N_OBJ)
    v16 = jnp.pad(vT, ((0, 0), (0, TW - N_OBJ)))
    m16 = jnp.pad(mask_table.T, ((0, 0), (0, TW - N_OBJ)))
    out2d = _sc_gather(v16, m16, idx)
    return out2d.reshape(3, PAD_LEN, BATCH).transpose(2, 1, 0)


# speedup vs baseline: 2.9099x; 1.0842x over previous
import functools

import jax
import jax.numpy as jnp
from jax import lax
from jax.experimental import pallas as pl
from jax.experimental.pallas import tpu as pltpu
from jax.experimental.pallas import tpu_sc as plsc

N_OBJ = 11
PAD_LEN = 4000
BATCH = 1024
NROW = PAD_LEN * 3
TW = 16

_info = plsc.get_sparse_core_info()
_NC = _info.num_cores
_NS = _info.num_subcores
_NW = _NC * _NS

_RC = 32
_N_CHUNKS = NROW // _RC
_CPW = -(-_N_CHUNKS // _NW)
_NIV = BATCH // 16
_IVB = 16


@functools.partial(
    pl.kernel,
    mesh=plsc.VectorSubcoreMesh(core_axis_name="c", subcore_axis_name="s"),
    out_type=jax.ShapeDtypeStruct((NROW, BATCH), jnp.float32),
    scratch_types=[
        pltpu.VMEM((BATCH,), jnp.int32),
        pltpu.VMEM((2, _RC, TW), jnp.float32),
        pltpu.VMEM((2, _RC, TW), jnp.float32),
        pltpu.VMEM((2, _RC, BATCH), jnp.float32),
        pltpu.SemaphoreType.DMA,
        pltpu.SemaphoreType.DMA,
        pltpu.SemaphoreType.DMA,
    ],
)
def _sc_gather(
    table_hbm, mask_hbm, idx_hbm, out_hbm,
    idx_v, tbl_v, msk_v, obuf, gsem, msem, ssem,
):
    wid = lax.axis_index("s") * _NC + lax.axis_index("c")
    pltpu.sync_copy(idx_hbm, idx_v)

    def do_chunk(c, b):
        gchunk = c * _NW + wid
        valid = gchunk < _N_CHUNKS
        row0 = pl.multiple_of(gchunk * _RC, _RC)

        @pl.when(valid & (c >= 2))
        def _wait_prev_scatter():
            pltpu.make_async_copy(
                obuf.at[b], out_hbm.at[pl.ds(0, _RC)], ssem
            ).wait()

        @pl.when(valid)
        def _work():
            pltpu.async_copy(
                table_hbm.at[pl.ds(row0, _RC)], tbl_v.at[b], gsem
            )
            mrow0 = pl.multiple_of(row0 - (row0 // PAD_LEN) * PAD_LEN, _RC)
            pltpu.async_copy(
                mask_hbm.at[pl.ds(mrow0, _RC)], msk_v.at[b], msem
            ).wait()
            pltpu.make_async_copy(
                table_hbm.at[pl.ds(row0, _RC)], tbl_v.at[b], gsem
            ).wait()

            def fold_body(r, _):
                tbl_v.at[b][r] = tbl_v.at[b][r] * msk_v.at[b][r]
                return _

            lax.fori_loop(0, _RC, fold_body, 0, unroll=False)

            for t in range(_NIV // _IVB):
                ivecs = [
                    idx_v[pl.ds((t * _IVB + j) * 16, 16)]
                    for j in range(_IVB)
                ]

                def row_body(r, _):
                    row = tbl_v.at[b][r]
                    for j in range(_IVB):
                        g = row.at[ivecs[j]].get(mode="promise_in_bounds")
                        obuf.at[b][r, pl.ds((t * _IVB + j) * 16, 16)] = g
                    return _

                lax.fori_loop(0, _RC, row_body, 0, unroll=False)
            pltpu.async_copy(
                obuf.at[b], out_hbm.at[pl.ds(row0, _RC)], ssem
            )

    def step(k, _):
        do_chunk(k * 2, 0)
        do_chunk(k * 2 + 1, 1)
        return _

    lax.fori_loop(0, _CPW // 2, step, 0, unroll=False)

    for b in range(2):
        pltpu.make_async_copy(
            obuf.at[b], out_hbm.at[pl.ds(0, _RC)], ssem
        ).wait()


def kernel(indices, v_table, mask_table):
    idx = indices.astype(jnp.int32)
    vT = v_table.transpose(2, 1, 0).reshape(NROW, N_OBJ)
    v16 = jnp.pad(vT, ((0, 0), (0, TW - N_OBJ)))
    m16 = jnp.pad(mask_table.T, ((0, 0), (0, TW - N_OBJ)))
    out2d = _sc_gather(v16, m16, idx)
    return out2d.reshape(3, PAD_LEN, BATCH).transpose(2, 1, 0)
